# Initial kernel scaffold; baseline (speedup 1.0000x reference)
import jax
import jax.numpy as jnp
from jax.experimental import pallas as pl


def kernel(node_memory, node_memory_ts, mailbox, mailbox_ts, edge_feats, w_ih, w_hh, b_ih, b_hh, time_w, time_b, all_nodes):
    raise NotImplementedError("write your pallas kernel here")


# speedup vs baseline: 1.5184x; 1.5184x over previous
import functools

import jax
import jax.numpy as jnp
from jax.experimental import pallas as pl
from jax.experimental.pallas import tpu as pltpu

N = 100000
L = 16384
E = L // 2
DM = 256
DE = 128
DMAIL = 2 * DM + DE
DT = 100
DTP = 128


def _gru_body(mem_ref, mail_ref, ts_ref, wm_ref, wt_ref, wh_ref,
              bi_ref, bh_ref, tw_ref, tb_ref, upd_ref):
    mem = mem_ref[...]
    dt = ts_ref[:, 1:2] - ts_ref[:, 0:1]
    tf = jnp.cos(dt * tw_ref[...] + tb_ref[...])
    gi = (jnp.dot(mail_ref[...], wm_ref[...], preferred_element_type=jnp.float32)
          + jnp.dot(tf, wt_ref[...], preferred_element_type=jnp.float32)
          + bi_ref[...])
    gh = jnp.dot(mem, wh_ref[...], preferred_element_type=jnp.float32) + bh_ref[...]
    r = jax.nn.sigmoid(gi[:, 0:DM] + gh[:, 0:DM])
    z = jax.nn.sigmoid(gi[:, DM:2 * DM] + gh[:, DM:2 * DM])
    n = jnp.tanh(gi[:, 2 * DM:] + r * gh[:, 2 * DM:])
    upd_ref[...] = (1.0 - z) * n + z * mem


def _tc_gru(mem_g, mail_g, ts_g, wm, wt, wh, bi, bh, tw, tb):
    BM = 1024
    grid = (L // BM,)
    return pl.pallas_call(
        _gru_body,
        grid=grid,
        in_specs=[
            pl.BlockSpec((BM, DM), lambda i: (i, 0)),
            pl.BlockSpec((BM, DMAIL), lambda i: (i, 0)),
            pl.BlockSpec((BM, 16), lambda i: (i, 0)),
            pl.BlockSpec((DMAIL, 3 * DM), lambda i: (0, 0)),
            pl.BlockSpec((DTP, 3 * DM), lambda i: (0, 0)),
            pl.BlockSpec((DM, 3 * DM), lambda i: (0, 0)),
            pl.BlockSpec((1, 3 * DM), lambda i: (0, 0)),
            pl.BlockSpec((1, 3 * DM), lambda i: (0, 0)),
            pl.BlockSpec((1, DTP), lambda i: (0, 0)),
            pl.BlockSpec((1, DTP), lambda i: (0, 0)),
        ],
        out_specs=pl.BlockSpec((BM, DM), lambda i: (i, 0)),
        out_shape=jax.ShapeDtypeStruct((L, DM), jnp.float32),
    )(mem_g, mail_g, ts_g, wm, wt, wh, bi, bh, tw, tb)


def kernel(node_memory, node_memory_ts, mailbox, mailbox_ts, edge_feats,
           w_ih, w_hh, b_ih, b_hh, time_w, time_b, all_nodes):
    w_ih_t = w_ih.T
    wm = w_ih_t[:DMAIL]
    wt = jnp.zeros((DTP, 3 * DM), jnp.float32).at[:DT].set(w_ih_t[DMAIL:])
    wh = w_hh.T
    bi = b_ih.reshape(1, -1)
    bh = b_hh.reshape(1, -1)
    tw = jnp.zeros((1, DTP), jnp.float32).at[0, :DT].set(time_w)
    tb = jnp.zeros((1, DTP), jnp.float32).at[0, :DT].set(time_b)

    mem_g = node_memory[all_nodes]
    mail_g = mailbox[all_nodes]
    mem_ts_g = node_memory_ts[all_nodes]
    mail_ts_g = mailbox_ts[all_nodes]
    ts_g = jnp.zeros((L, 16), jnp.float32)
    ts_g = ts_g.at[:, 0].set(mem_ts_g).at[:, 1].set(mail_ts_g)

    updated = _tc_gru(mem_g, mail_g, ts_g, wm, wt, wh, bi, bh, tw, tb)

    node_memory_new = node_memory.at[all_nodes].set(updated, mode="drop")
    node_memory_ts_new = node_memory_ts.at[all_nodes].set(mail_ts_g, mode="drop")

    src = all_nodes[:E]
    dst = all_nodes[E:]
    nid = jnp.stack([src, dst], axis=1).reshape(-1)
    pos = jnp.arange(L, dtype=jnp.int32)
    maxpos = jnp.full((N,), -1, jnp.int32).at[nid].max(pos, mode="drop")
    winner = maxpos[nid] == pos
    tgt = jnp.where(winner, nid, N)
    mem_src = updated[:E]
    mem_dst = updated[E:]
    src_mail = jnp.concatenate([mem_src, mem_dst, edge_feats], axis=1)
    dst_mail = jnp.concatenate([mem_dst, mem_src, edge_feats], axis=1)
    mail_new = jnp.stack([src_mail, dst_mail], axis=1).reshape(-1, DMAIL)
    mailbox_new = mailbox.at[tgt].set(mail_new, mode="drop")
    mailbox_ts_new = mailbox_ts.at[tgt].set(mail_ts_g, mode="drop")

    return (updated, node_memory_new, node_memory_ts_new,
            mailbox_new, mailbox_ts_new)


# speedup vs baseline: 2.5392x; 1.6723x over previous
import functools

import jax
import jax.numpy as jnp
from jax import lax
from jax.experimental import pallas as pl
from jax.experimental.pallas import tpu as pltpu
from jax.experimental.pallas import tpu_sc as plsc

N = 100000
L = 16384
E = L // 2
DM = 256
DE = 128
DMAIL = 2 * DM + DE
DT = 100
DTP = 128

NC = 2
NS = 16
NW = NC * NS
RPW = L // NW
GC = 64
SLICE = 3128
NPAD = NW * SLICE
LANES = 16

_mesh = plsc.VectorSubcoreMesh(core_axis_name="c", subcore_axis_name="s")


def _interleaved(an_v, j, h, lane):
    posv = 32 * j + 16 * h + lane
    half = lax.shift_right_logical(posv, 1)
    even = (posv & 1) == 0
    kidx = half + jnp.where(even, 0, E)
    nidv = plsc.load_gather(an_v, [kidx])
    return nidv, posv


def _sc_gather_body(nm_hbm, mb_hbm, ts_hbm, an_hbm,
                    mem_g, mail_g, ts_g, maxpos,
                    an_v, mem_b, mail_b, ts_b, tab_v, sh_v,
                    sem_in, sem_out):
    wid = lax.axis_index("s") * NC + lax.axis_index("c")
    base = wid * RPW
    lane = lax.iota(jnp.int32, LANES)

    pltpu.sync_copy(an_hbm, an_v)

    def chunk(t, _):
        off = base + t * GC
        idx = an_v.at[pl.ds(off, GC)]
        cm = pltpu.async_copy(nm_hbm.at[idx], mem_b, sem_in)
        cb = pltpu.async_copy(mb_hbm.at[idx], mail_b, sem_in)
        ct = pltpu.async_copy(ts_hbm.at[idx], ts_b, sem_in)
        cm.wait()
        cb.wait()
        ct.wait()
        pltpu.async_copy(mem_b, mem_g.at[pl.ds(off, GC)], sem_out).wait()
        pltpu.async_copy(mail_b, mail_g.at[pl.ds(off, GC)], sem_out).wait()
        pltpu.async_copy(ts_b, ts_g.at[pl.ds(off, GC)], sem_out).wait()
        return 0

    lax.fori_loop(0, RPW // GC, chunk, 0)

    wlo = wid * SLICE

    def clear(i, _):
        tab_v[pl.ds(i * LANES, LANES)] = jnp.full((LANES,), -1, jnp.int32)
        return 0

    lax.fori_loop(0, SLICE // LANES + 1, clear, 0)

    def scan(j, _):
        for h in (0, 1):
            nidv, posv = _interleaved(an_v, j, h, lane)
            key = (nidv << 14) | posv
            sk, sv = plsc.sort_key_val(key, posv)
            nid_s = lax.shift_right_logical(sk, 14)
            sh_v[...] = nid_s
            nxt = plsc.load_gather(sh_v, [jnp.minimum(lane + 1, LANES - 1)])
            is_end = (lane == LANES - 1) | (nid_s != nxt)
            m = is_end & (nid_s >= wlo) & (nid_s < wlo + SLICE)
            plsc.store_scatter(tab_v, [nid_s - wlo], sv, mask=m)
        return 0

    lax.fori_loop(0, L // 32, scan, 0)
    pltpu.sync_copy(tab_v.at[pl.ds(0, SLICE)], maxpos.at[pl.ds(wlo, SLICE)])


@functools.partial(
    pl.kernel,
    out_type=(
        jax.ShapeDtypeStruct((L, DM), jnp.float32),
        jax.ShapeDtypeStruct((L, DMAIL), jnp.float32),
        jax.ShapeDtypeStruct((L, 128), jnp.float32),
        jax.ShapeDtypeStruct((NPAD,), jnp.int32),
    ),
    mesh=_mesh,
    compiler_params=pltpu.CompilerParams(needs_layout_passes=False),
    scratch_types=[
        pltpu.VMEM((L,), jnp.int32),
        pltpu.VMEM((GC, DM), jnp.float32),
        pltpu.VMEM((GC, DMAIL), jnp.float32),
        pltpu.VMEM((GC, 128), jnp.float32),
        pltpu.VMEM((SLICE + LANES, ), jnp.int32),
        pltpu.VMEM((LANES,), jnp.int32),
        pltpu.SemaphoreType.DMA,
        pltpu.SemaphoreType.DMA,
    ],
)
def _sc_gather(*refs):
    _sc_gather_body(*refs)


def _gru_body(mem_ref, mail_ref, ts_ref, wm_ref, wt_ref, wh_ref,
              bi_ref, bh_ref, tw_ref, tb_ref, upd_ref):
    mem = mem_ref[...]
    dt = ts_ref[:, 1:2] - ts_ref[:, 0:1]
    tf = jnp.cos(dt * tw_ref[...] + tb_ref[...])
    gi = (jnp.dot(mail_ref[...], wm_ref[...], preferred_element_type=jnp.float32)
          + jnp.dot(tf, wt_ref[...], preferred_element_type=jnp.float32)
          + bi_ref[...])
    gh = jnp.dot(mem, wh_ref[...], preferred_element_type=jnp.float32) + bh_ref[...]
    r = jax.nn.sigmoid(gi[:, 0:DM] + gh[:, 0:DM])
    z = jax.nn.sigmoid(gi[:, DM:2 * DM] + gh[:, DM:2 * DM])
    n = jnp.tanh(gi[:, 2 * DM:] + r * gh[:, 2 * DM:])
    upd_ref[...] = (1.0 - z) * n + z * mem


def _tc_gru(mem_g, mail_g, ts_g, wm, wt, wh, bi, bh, tw, tb):
    BM = 1024
    return pl.pallas_call(
        _gru_body,
        grid=(L // BM,),
        in_specs=[
            pl.BlockSpec((BM, DM), lambda i: (i, 0)),
            pl.BlockSpec((BM, DMAIL), lambda i: (i, 0)),
            pl.BlockSpec((BM, 128), lambda i: (i, 0)),
            pl.BlockSpec((DMAIL, 3 * DM), lambda i: (0, 0)),
            pl.BlockSpec((DTP, 3 * DM), lambda i: (0, 0)),
            pl.BlockSpec((DM, 3 * DM), lambda i: (0, 0)),
            pl.BlockSpec((1, 3 * DM), lambda i: (0, 0)),
            pl.BlockSpec((1, 3 * DM), lambda i: (0, 0)),
            pl.BlockSpec((1, DTP), lambda i: (0, 0)),
            pl.BlockSpec((1, DTP), lambda i: (0, 0)),
        ],
        out_specs=pl.BlockSpec((BM, DM), lambda i: (i, 0)),
        out_shape=jax.ShapeDtypeStruct((L, DM), jnp.float32),
    )(mem_g, mail_g, ts_g, wm, wt, wh, bi, bh, tw, tb)


def _mail_body(us_ref, ud_ref, ef_ref, ms_ref, md_ref):
    s = us_ref[...]
    d = ud_ref[...]
    e = ef_ref[...]
    ms_ref[:, 0:DM] = s
    ms_ref[:, DM:2 * DM] = d
    ms_ref[:, 2 * DM:] = e
    md_ref[:, 0:DM] = d
    md_ref[:, DM:2 * DM] = s
    md_ref[:, 2 * DM:] = e


def _tc_mail(updated, edge_feats):
    BE = 1024
    nb = E // BE
    return pl.pallas_call(
        _mail_body,
        grid=(nb,),
        in_specs=[
            pl.BlockSpec((BE, DM), lambda i: (i, 0)),
            pl.BlockSpec((BE, DM), lambda i: (i + nb, 0)),
            pl.BlockSpec((BE, DE), lambda i: (i, 0)),
        ],
        out_specs=[
            pl.BlockSpec((BE, DMAIL), lambda i: (i, 0)),
            pl.BlockSpec((BE, DMAIL), lambda i: (i, 0)),
        ],
        out_shape=[
            jax.ShapeDtypeStruct((E, DMAIL), jnp.float32),
            jax.ShapeDtypeStruct((E, DMAIL), jnp.float32),
        ],
    )(updated, updated, edge_feats)


def _sc_scatter_body(nm_ref, nmts_ref, mb_ref, mbts_ref,
                     upd_hbm, msrc_hbm, mdst_hbm, mts_hbm, an_hbm, maxpos_hbm,
                     an_v, mts_v, tab_v, nmts_v, mbts_v,
                     memk, pe, po, rb256, rb640, sem_g, sem_s):
    wid = lax.axis_index("s") * NC + lax.axis_index("c")
    wlo = wid * SLICE
    whi = wlo + SLICE
    lane = lax.iota(jnp.int32, LANES)

    pltpu.sync_copy(an_hbm, an_v)
    pltpu.sync_copy(mts_hbm, mts_v)
    pltpu.sync_copy(maxpos_hbm.at[pl.ds(wlo, SLICE)], tab_v)
    pltpu.sync_copy(nmts_ref.at[pl.ds(wlo, SLICE)], nmts_v)
    pltpu.sync_copy(mbts_ref.at[pl.ds(wlo, SLICE)], mbts_v)

    def scan_mem(j, cnt):
        kv = j * LANES + lane
        u = an_v[pl.ds(j * LANES, LANES)]
        m = (u >= wlo) & (u < whi)
        plsc.store_compressed(memk.at[pl.ds(cnt, LANES)], kv, mask=m)
        return cnt + plsc.all_reduce_population_count(m)[0]

    cnt_k = lax.fori_loop(0, L // LANES, scan_mem, jnp.int32(0))

    def scan_win(j, cnts):
        ce, co = cnts
        for h in (0, 1):
            nidv, posv = _interleaved(an_v, j, h, lane)
            inr = (nidv >= wlo) & (nidv < whi)
            t = plsc.load_gather(tab_v, [jnp.where(inr, nidv - wlo, 0)])
            win = inr & (t == posv)
            even = (posv & 1) == 0
            me = win & even
            mo = win & (~even)
            plsc.store_compressed(pe.at[pl.ds(ce, LANES)], posv, mask=me)
            ce = ce + plsc.all_reduce_population_count(me)[0]
            plsc.store_compressed(po.at[pl.ds(co, LANES)], posv, mask=mo)
            co = co + plsc.all_reduce_population_count(mo)[0]
        return (ce, co)

    cnt_e, cnt_o = lax.fori_loop(0, L // 32, scan_win,
                                 (jnp.int32(0), jnp.int32(0)))

    def _pad(lst, cnt):
        @pl.when(cnt > 0)
        def _():
            lst[pl.ds(cnt, LANES)] = plsc.load_gather(
                lst, [jnp.zeros((LANES,), jnp.int32)])

    _pad(memk, cnt_k)
    _pad(pe, cnt_e)
    _pad(po, cnt_o)

    trips_k = (cnt_k + LANES - 1) // LANES
    trips_e = (cnt_e + LANES - 1) // LANES
    trips_o = (cnt_o + LANES - 1) // LANES

    def ts_mem(t, _):
        kv = memk[pl.ds(t * LANES, LANES)]
        uv = plsc.load_gather(an_v, [kv])
        tv = plsc.load_gather(mts_v, [kv])
        plsc.store_scatter(nmts_v, [uv - wlo], tv)
        return 0

    lax.fori_loop(0, trips_k, ts_mem, 0)

    def ts_mail(off):
        def body(t, _):
            pv = pe[pl.ds(t * LANES, LANES)] if off == 0 else po[pl.ds(t * LANES, LANES)]
            ev = lax.shift_right_logical(pv, 1)
            uv = plsc.load_gather(an_v, [ev + off])
            tv = plsc.load_gather(mts_v, [pv])
            plsc.store_scatter(mbts_v, [uv - wlo], tv)
            return 0
        return body

    lax.fori_loop(0, trips_e, ts_mail(0), 0)
    lax.fori_loop(0, trips_o, ts_mail(E), 0)

    pltpu.sync_copy(nmts_v, nmts_ref.at[pl.ds(wlo, SLICE)])
    pltpu.sync_copy(mbts_v, mbts_ref.at[pl.ds(wlo, SLICE)])

    def row_mem(t, _):
        kv = memk[pl.ds(t * LANES, LANES)]
        uv = plsc.load_gather(an_v, [kv])
        pltpu.async_copy(upd_hbm.at[kv], rb256, sem_g).wait()
        pltpu.async_copy(rb256, nm_ref.at[uv], sem_s).wait()
        return 0

    lax.fori_loop(0, trips_k, row_mem, 0)

    def row_mail(src_hbm, lst, off):
        def body(t, _):
            pv = lst[pl.ds(t * LANES, LANES)]
            ev = lax.shift_right_logical(pv, 1)
            uv = plsc.load_gather(an_v, [ev + off])
            pltpu.async_copy(src_hbm.at[ev], rb640, sem_g).wait()
            pltpu.async_copy(rb640, mb_ref.at[uv], sem_s).wait()
            return 0
        return body

    lax.fori_loop(0, trips_e, row_mail(msrc_hbm, pe, 0), 0)
    lax.fori_loop(0, trips_o, row_mail(mdst_hbm, po, E), 0)


@functools.partial(
    pl.kernel,
    out_type=(),
    mesh=_mesh,
    compiler_params=pltpu.CompilerParams(needs_layout_passes=False),
    scratch_types=[
        pltpu.VMEM((L,), jnp.int32),
        pltpu.VMEM((L,), jnp.float32),
        pltpu.VMEM((SLICE,), jnp.int32),
        pltpu.VMEM((SLICE,), jnp.float32),
        pltpu.VMEM((SLICE,), jnp.float32),
        pltpu.VMEM((L + LANES,), jnp.int32),
        pltpu.VMEM((E + LANES,), jnp.int32),
        pltpu.VMEM((E + LANES,), jnp.int32),
        pltpu.VMEM((LANES, DM), jnp.float32),
        pltpu.VMEM((LANES, DMAIL), jnp.float32),
        pltpu.SemaphoreType.DMA,
        pltpu.SemaphoreType.DMA,
    ],
)
def _sc_scatter(*refs):
    _sc_scatter_body(*refs)


def kernel(node_memory, node_memory_ts, mailbox, mailbox_ts, edge_feats,
           w_ih, w_hh, b_ih, b_hh, time_w, time_b, all_nodes):
    w_ih_t = w_ih.T
    wm = w_ih_t[:DMAIL]
    wt = jnp.zeros((DTP, 3 * DM), jnp.float32).at[:DT].set(w_ih_t[DMAIL:])
    wh = w_hh.T
    bi = b_ih.reshape(1, -1)
    bh = b_hh.reshape(1, -1)
    tw = jnp.zeros((1, DTP), jnp.float32).at[0, :DT].set(time_w)
    tb = jnp.zeros((1, DTP), jnp.float32).at[0, :DT].set(time_b)
    ts_tab = jnp.zeros((N, 128), jnp.float32)
    ts_tab = ts_tab.at[:, 0].set(node_memory_ts).at[:, 1].set(mailbox_ts)

    mem_g, mail_g, ts_g, maxpos = _sc_gather(node_memory, mailbox, ts_tab,
                                             all_nodes)
    updated = _tc_gru(mem_g, mail_g, ts_g, wm, wt, wh, bi, bh, tw, tb)
    mail_src, mail_dst = _tc_mail(updated, edge_feats)
    mts_g = ts_g[:, 1]

    nm_ref = jax.new_ref(node_memory)
    mb_ref = jax.new_ref(mailbox)
    nmts_ref = jax.new_ref(jnp.pad(node_memory_ts, (0, NPAD - N)))
    mbts_ref = jax.new_ref(jnp.pad(mailbox_ts, (0, NPAD - N)))
    _sc_scatter(nm_ref, nmts_ref, mb_ref, mbts_ref,
                updated, mail_src, mail_dst, mts_g, all_nodes, maxpos)

    return (updated, nm_ref[...], nmts_ref[...][:N],
            mb_ref[...], mbts_ref[...][:N])


# speedup vs baseline: 4.3343x; 1.7069x over previous
import functools

import jax
import jax.numpy as jnp
from jax import lax
from jax.experimental import pallas as pl
from jax.experimental.pallas import tpu as pltpu
from jax.experimental.pallas import tpu_sc as plsc

N = 100000
L = 16384
E = L // 2
DM = 256
DE = 128
DMAIL = 2 * DM + DE
DT = 100
DTP = 128

NC = 2
NS = 16
NW = NC * NS
RPW = L // NW
GC = 32
SLICE = 3128
NPAD = NW * SLICE
TSR = NPAD // 128
LANES = 16
BMEM = 128
BMB = 48

_mesh = plsc.VectorSubcoreMesh(core_axis_name="c", subcore_axis_name="s")
_sc_params = pltpu.CompilerParams(needs_layout_passes=False)


def _worker_id():
    return lax.axis_index("s") * NC + lax.axis_index("c")


def _interleaved(an_v, j, h, lane):
    posv = 32 * j + 16 * h + lane
    half = lax.shift_right_logical(posv, 1)
    even = (posv & 1) == 0
    kidx = half + jnp.where(even, 0, E)
    nidv = plsc.load_gather(an_v, [kidx])
    return nidv, posv


def _sc_gather_body(nm_hbm, mb_hbm, tsa_hbm, tsb_hbm, an_hbm,
                    mem_g, mail_g, ts2,
                    an_v, rowidx_v, mem_b0, mem_b1, mail_b0, mail_b1,
                    ts_b, ts_stage,
                    sem_g0, sem_g1, sem_o0, sem_o1, sem_ts):
    wid = _worker_id()
    base = wid * RPW
    lane = lax.iota(jnp.int32, LANES)

    pltpu.sync_copy(an_hbm, an_v)

    def rowbuild(g, _):
        ids = an_v[pl.ds(base + g * LANES, LANES)]
        rowidx_v[pl.ds(g * LANES, LANES)] = lax.shift_right_logical(ids, 7)
        return 0

    lax.fori_loop(0, RPW // LANES, rowbuild, 0)

    HALF = 256
    for ti, tref in ((0, tsa_hbm), (1, tsb_hbm)):
        for hf in (0, 1):
            pltpu.async_copy(tref.at[rowidx_v.at[pl.ds(hf * HALF, HALF)]],
                             ts_b, sem_ts).wait()

            def extract(g, _):
                ids = an_v[pl.ds(base + hf * HALF + g * LANES, LANES)]
                colv = ids & 127
                v = plsc.load_gather(ts_b, [g * LANES + lane, colv])
                ts_stage[ti, pl.ds(hf * HALF + g * LANES, LANES)] = v
                return 0

            lax.fori_loop(0, HALF // LANES, extract, 0)

    pltpu.sync_copy(ts_stage, ts2.at[:, pl.ds(base, RPW)])

    mem_bufs = (mem_b0, mem_b1)
    mail_bufs = (mail_b0, mail_b1)
    sems_g = (sem_g0, sem_g1)
    sems_o = (sem_o0, sem_o1)
    nchunks = RPW // GC

    def issue(t, par):
        off = base + t * GC
        idx = an_v.at[pl.ds(off, GC)]
        pltpu.async_copy(nm_hbm.at[idx], mem_bufs[par], sems_g[par])
        pltpu.async_copy(mb_hbm.at[idx], mail_bufs[par], sems_g[par])

    for par in (0, 1):
        issue(par, par)

    def chunk(tp, _):
        for par in (0, 1):
            t = 2 * tp + par
            off = base + t * GC
            idx = an_v.at[pl.ds(off, GC)]
            pltpu.make_async_copy(nm_hbm.at[idx], mem_bufs[par],
                                  sems_g[par]).wait()
            pltpu.make_async_copy(mb_hbm.at[idx], mail_bufs[par],
                                  sems_g[par]).wait()
            cm = pltpu.async_copy(mem_bufs[par], mem_g.at[pl.ds(off, GC)],
                                  sems_o[par])
            cb = pltpu.async_copy(mail_bufs[par], mail_g.at[pl.ds(off, GC)],
                                  sems_o[par])
            cm.wait()
            cb.wait()

            @pl.when(t + 2 < nchunks)
            def _():
                issue(t + 2, par)

        return 0

    lax.fori_loop(0, nchunks // 2, chunk, 0)


@functools.partial(
    pl.kernel,
    out_type=(
        jax.ShapeDtypeStruct((L, DM), jnp.float32),
        jax.ShapeDtypeStruct((L, DMAIL), jnp.float32),
        jax.ShapeDtypeStruct((2, L), jnp.float32),
    ),
    mesh=_mesh,
    compiler_params=_sc_params,
    scratch_types=[
        pltpu.VMEM((L,), jnp.int32),
        pltpu.VMEM((RPW,), jnp.int32),
        pltpu.VMEM((GC, DM), jnp.float32),
        pltpu.VMEM((GC, DM), jnp.float32),
        pltpu.VMEM((GC, DMAIL), jnp.float32),
        pltpu.VMEM((GC, DMAIL), jnp.float32),
        pltpu.VMEM((256, 128), jnp.float32),
        pltpu.VMEM((2, RPW), jnp.float32),
        pltpu.SemaphoreType.DMA,
        pltpu.SemaphoreType.DMA,
        pltpu.SemaphoreType.DMA,
        pltpu.SemaphoreType.DMA,
        pltpu.SemaphoreType.DMA,
    ],
)
def _sc_gather(*refs):
    _sc_gather_body(*refs)


def _sc_winner_body(an_hbm, maxpos, an_v, tab_v, sh_v):
    wid = _worker_id()
    wlo = wid * SLICE
    lane = lax.iota(jnp.int32, LANES)

    pltpu.sync_copy(an_hbm, an_v)

    def clear(i, _):
        tab_v[pl.ds(i * LANES, LANES)] = jnp.full((LANES,), -1, jnp.int32)
        return 0

    lax.fori_loop(0, SLICE // LANES + 1, clear, 0)

    def scan(j, _):
        for h in (0, 1):
            nidv, posv = _interleaved(an_v, j, h, lane)
            key = (nidv << 14) | posv
            sk, sv = plsc.sort_key_val(key, posv)
            nid_s = lax.shift_right_logical(sk, 14)
            sh_v[...] = nid_s
            nxt = plsc.load_gather(sh_v, [jnp.minimum(lane + 1, LANES - 1)])
            is_end = (lane == LANES - 1) | (nid_s != nxt)
            m = is_end & (nid_s >= wlo) & (nid_s < wlo + SLICE)
            plsc.store_scatter(tab_v, [nid_s - wlo], sv, mask=m)
        return 0

    lax.fori_loop(0, L // 32, scan, 0)
    pltpu.sync_copy(tab_v.at[pl.ds(0, SLICE)], maxpos.at[pl.ds(wlo, SLICE)])


@functools.partial(
    pl.kernel,
    out_type=jax.ShapeDtypeStruct((NPAD,), jnp.int32),
    mesh=_mesh,
    compiler_params=_sc_params,
    scratch_types=[
        pltpu.VMEM((L,), jnp.int32),
        pltpu.VMEM((SLICE + LANES,), jnp.int32),
        pltpu.VMEM((LANES,), jnp.int32),
    ],
)
def _sc_winner(*refs):
    _sc_winner_body(*refs)


def _gru_body(mem_ref, mail_ref, tsm_ref, tsb_ref, wm_ref, wt_ref, wh_ref,
              bi_ref, bh_ref, tw_ref, tb_ref, upd_ref):
    mem = mem_ref[...]
    dt = tsb_ref[...] - tsm_ref[...]
    tf = jnp.cos(dt * tw_ref[...] + tb_ref[...])
    gi = (jnp.dot(mail_ref[...], wm_ref[...], preferred_element_type=jnp.float32)
          + jnp.dot(tf, wt_ref[...], preferred_element_type=jnp.float32)
          + bi_ref[...])
    gh = jnp.dot(mem, wh_ref[...], preferred_element_type=jnp.float32) + bh_ref[...]
    r = jax.nn.sigmoid(gi[:, 0:DM] + gh[:, 0:DM])
    z = jax.nn.sigmoid(gi[:, DM:2 * DM] + gh[:, DM:2 * DM])
    n = jnp.tanh(gi[:, 2 * DM:] + r * gh[:, 2 * DM:])
    upd_ref[...] = (1.0 - z) * n + z * mem


def _tc_gru(mem_g, mail_g, tsm, tsb, wm, wt, wh, bi, bh, tw, tb):
    BM = 1024
    return pl.pallas_call(
        _gru_body,
        grid=(L // BM,),
        in_specs=[
            pl.BlockSpec((BM, DM), lambda i: (i, 0)),
            pl.BlockSpec((BM, DMAIL), lambda i: (i, 0)),
            pl.BlockSpec((BM, 1), lambda i: (i, 0)),
            pl.BlockSpec((BM, 1), lambda i: (i, 0)),
            pl.BlockSpec((DMAIL, 3 * DM), lambda i: (0, 0)),
            pl.BlockSpec((DTP, 3 * DM), lambda i: (0, 0)),
            pl.BlockSpec((DM, 3 * DM), lambda i: (0, 0)),
            pl.BlockSpec((1, 3 * DM), lambda i: (0, 0)),
            pl.BlockSpec((1, 3 * DM), lambda i: (0, 0)),
            pl.BlockSpec((1, DTP), lambda i: (0, 0)),
            pl.BlockSpec((1, DTP), lambda i: (0, 0)),
        ],
        out_specs=pl.BlockSpec((BM, DM), lambda i: (i, 0)),
        out_shape=jax.ShapeDtypeStruct((L, DM), jnp.float32),
    )(mem_g, mail_g, tsm, tsb, wm, wt, wh, bi, bh, tw, tb)


def _mail_body(us_ref, ud_ref, ef_ref, ms_ref, md_ref):
    s = us_ref[...]
    d = ud_ref[...]
    e = ef_ref[...]
    ms_ref[:, 0:DM] = s
    ms_ref[:, DM:2 * DM] = d
    ms_ref[:, 2 * DM:] = e
    md_ref[:, 0:DM] = d
    md_ref[:, DM:2 * DM] = s
    md_ref[:, 2 * DM:] = e


def _tc_mail(updated, edge_feats):
    BE = 1024
    nb = E // BE
    return pl.pallas_call(
        _mail_body,
        grid=(nb,),
        in_specs=[
            pl.BlockSpec((BE, DM), lambda i: (i, 0)),
            pl.BlockSpec((BE, DM), lambda i: (i + nb, 0)),
            pl.BlockSpec((BE, DE), lambda i: (i, 0)),
        ],
        out_specs=[
            pl.BlockSpec((BE, DMAIL), lambda i: (i, 0)),
            pl.BlockSpec((BE, DMAIL), lambda i: (i, 0)),
        ],
        out_shape=[
            jax.ShapeDtypeStruct((E, DMAIL), jnp.float32),
            jax.ShapeDtypeStruct((E, DMAIL), jnp.float32),
        ],
    )(updated, updated, edge_feats)


def _pad_list(lst, cnt, batch):
    @pl.when(cnt > 0)
    def _():
        rep = plsc.load_gather(lst, [jnp.zeros((LANES,), jnp.int32)])
        for i in range(batch // LANES):
            lst[pl.ds(cnt + i * LANES, LANES)] = rep


def _scatter_rows(lst, cnt, batch, src_hbm, dst_ref, an_v, off,
                  rbufs, ubufs, ebufs, sems_g, sems_s, via_edge):
    trips = (cnt + batch - 1) // batch

    def src_idx(t, par):
        if via_edge:
            return ebufs[par]
        return lst.at[pl.ds(t * batch, batch)]

    def prep_and_issue(t, par):
        for i in range(batch // LANES):
            v = lst[pl.ds(t * batch + i * LANES, LANES)]
            if via_edge:
                ev = lax.shift_right_logical(v, 1)
                ebufs[par][pl.ds(i * LANES, LANES)] = ev
            else:
                ev = v
            uv = plsc.load_gather(an_v, [ev + off])
            ubufs[par][pl.ds(i * LANES, LANES)] = uv
        pltpu.async_copy(src_hbm.at[src_idx(t, par)], rbufs[par], sems_g[par])

    @pl.when(trips > 0)
    def _():
        prep_and_issue(0, 0)

    def body(tp, _):
        for par in (0, 1):
            t = 2 * tp + par

            @pl.when(t + 1 < trips)
            def _():
                prep_and_issue(t + 1, 1 - par)

            @pl.when(t < trips)
            def _():
                pltpu.make_async_copy(src_hbm.at[src_idx(t, par)],
                                      rbufs[par], sems_g[par]).wait()
                pltpu.async_copy(rbufs[par], dst_ref.at[ubufs[par]],
                                 sems_s[par]).wait()

        return 0

    lax.fori_loop(0, (trips + 1) // 2, body, 0)


def _sc_cmem_body(nm_ref, nmts_ref, upd_hbm, mts_hbm, an_hbm,
                  an_v, mts_v, nmts_v, memk,
                  rb0, rb1, iu0, iu1,
                  sem_g0, sem_g1, sem_s0, sem_s1):
    wid = _worker_id()
    wlo = wid * SLICE
    whi = wlo + SLICE
    lane = lax.iota(jnp.int32, LANES)

    pltpu.sync_copy(an_hbm, an_v)
    pltpu.sync_copy(mts_hbm, mts_v)
    pltpu.sync_copy(nmts_ref.at[pl.ds(wlo, SLICE)], nmts_v)

    def scan_mem(j, cnt):
        kv = j * LANES + lane
        u = an_v[pl.ds(j * LANES, LANES)]
        m = (u >= wlo) & (u < whi)
        plsc.store_compressed(memk.at[pl.ds(cnt, LANES)], kv, mask=m)
        return cnt + plsc.all_reduce_population_count(m)[0]

    cnt_k = lax.fori_loop(0, L // LANES, scan_mem, jnp.int32(0))
    _pad_list(memk, cnt_k, BMEM)

    def ts_mem(t, _):
        kv = memk[pl.ds(t * LANES, LANES)]
        uv = plsc.load_gather(an_v, [kv])
        tv = plsc.load_gather(mts_v, [kv])
        plsc.store_scatter(nmts_v, [uv - wlo], tv)
        return 0

    lax.fori_loop(0, (cnt_k + LANES - 1) // LANES, ts_mem, 0)
    pltpu.sync_copy(nmts_v, nmts_ref.at[pl.ds(wlo, SLICE)])

    _scatter_rows(memk, cnt_k, BMEM, upd_hbm, nm_ref, an_v, 0,
                  (rb0, rb1), (iu0, iu1), None,
                  (sem_g0, sem_g1), (sem_s0, sem_s1), via_edge=False)


@functools.partial(
    pl.kernel,
    out_type=(),
    mesh=_mesh,
    compiler_params=_sc_params,
    scratch_types=[
        pltpu.VMEM((L,), jnp.int32),
        pltpu.VMEM((L,), jnp.float32),
        pltpu.VMEM((SLICE,), jnp.float32),
        pltpu.VMEM((L + BMEM,), jnp.int32),
        pltpu.VMEM((BMEM, DM), jnp.float32),
        pltpu.VMEM((BMEM, DM), jnp.float32),
        pltpu.VMEM((BMEM,), jnp.int32),
        pltpu.VMEM((BMEM,), jnp.int32),
        pltpu.SemaphoreType.DMA,
        pltpu.SemaphoreType.DMA,
        pltpu.SemaphoreType.DMA,
        pltpu.SemaphoreType.DMA,
    ],
)
def _sc_cmem(*refs):
    _sc_cmem_body(*refs)


def _sc_cmail_body(mb_ref, mbts_ref, msrc_hbm, mdst_hbm, mts_hbm, an_hbm,
                   maxpos_hbm,
                   an_v, mts_v, tab_v, mbts_v, pe, po,
                   rb0, rb1, iu0, iu1, ie0, ie1,
                   sem_g0, sem_g1, sem_s0, sem_s1):
    wid = _worker_id()
    wlo = wid * SLICE
    whi = wlo + SLICE
    lane = lax.iota(jnp.int32, LANES)

    pltpu.sync_copy(an_hbm, an_v)
    pltpu.sync_copy(mts_hbm, mts_v)
    pltpu.sync_copy(maxpos_hbm.at[pl.ds(wlo, SLICE)], tab_v)
    pltpu.sync_copy(mbts_ref.at[pl.ds(wlo, SLICE)], mbts_v)

    def scan_win(j, cnts):
        ce, co = cnts
        for h in (0, 1):
            nidv, posv = _interleaved(an_v, j, h, lane)
            inr = (nidv >= wlo) & (nidv < whi)
            t = plsc.load_gather(tab_v, [jnp.where(inr, nidv - wlo, 0)])
            win = inr & (t == posv)
            even = (posv & 1) == 0
            me = win & even
            mo = win & (~even)
            plsc.store_compressed(pe.at[pl.ds(ce, LANES)], posv, mask=me)
            ce = ce + plsc.all_reduce_population_count(me)[0]
            plsc.store_compressed(po.at[pl.ds(co, LANES)], posv, mask=mo)
            co = co + plsc.all_reduce_population_count(mo)[0]
        return (ce, co)

    cnt_e, cnt_o = lax.fori_loop(0, L // 32, scan_win,
                                 (jnp.int32(0), jnp.int32(0)))
    _pad_list(pe, cnt_e, BMB)
    _pad_list(po, cnt_o, BMB)

    def ts_mail(lst, off):
        def body(t, _):
            pv = lst[pl.ds(t * LANES, LANES)]
            ev = lax.shift_right_logical(pv, 1)
            uv = plsc.load_gather(an_v, [ev + off])
            tv = plsc.load_gather(mts_v, [pv])
            plsc.store_scatter(mbts_v, [uv - wlo], tv)
            return 0
        return body

    lax.fori_loop(0, (cnt_e + LANES - 1) // LANES, ts_mail(pe, 0), 0)
    lax.fori_loop(0, (cnt_o + LANES - 1) // LANES, ts_mail(po, E), 0)
    pltpu.sync_copy(mbts_v, mbts_ref.at[pl.ds(wlo, SLICE)])

    _scatter_rows(pe, cnt_e, BMB, msrc_hbm, mb_ref, an_v, 0,
                  (rb0, rb1), (iu0, iu1), (ie0, ie1),
                  (sem_g0, sem_g1), (sem_s0, sem_s1), via_edge=True)
    _scatter_rows(po, cnt_o, BMB, mdst_hbm, mb_ref, an_v, E,
                  (rb0, rb1), (iu0, iu1), (ie0, ie1),
                  (sem_g0, sem_g1), (sem_s0, sem_s1), via_edge=True)


@functools.partial(
    pl.kernel,
    out_type=(),
    mesh=_mesh,
    compiler_params=_sc_params,
    scratch_types=[
        pltpu.VMEM((L,), jnp.int32),
        pltpu.VMEM((L,), jnp.float32),
        pltpu.VMEM((SLICE,), jnp.int32),
        pltpu.VMEM((SLICE,), jnp.float32),
        pltpu.VMEM((E + BMB,), jnp.int32),
        pltpu.VMEM((E + BMB,), jnp.int32),
        pltpu.VMEM((BMB, DMAIL), jnp.float32),
        pltpu.VMEM((BMB, DMAIL), jnp.float32),
        pltpu.VMEM((BMB,), jnp.int32),
        pltpu.VMEM((BMB,), jnp.int32),
        pltpu.VMEM((BMB,), jnp.int32),
        pltpu.VMEM((BMB,), jnp.int32),
        pltpu.SemaphoreType.DMA,
        pltpu.SemaphoreType.DMA,
        pltpu.SemaphoreType.DMA,
        pltpu.SemaphoreType.DMA,
    ],
)
def _sc_cmail(*refs):
    _sc_cmail_body(*refs)


def kernel(node_memory, node_memory_ts, mailbox, mailbox_ts, edge_feats,
           w_ih, w_hh, b_ih, b_hh, time_w, time_b, all_nodes):
    w_ih_t = w_ih.T
    wm = w_ih_t[:DMAIL]
    wt = jnp.zeros((DTP, 3 * DM), jnp.float32).at[:DT].set(w_ih_t[DMAIL:])
    wh = w_hh.T
    bi = b_ih.reshape(1, -1)
    bh = b_hh.reshape(1, -1)
    tw = jnp.zeros((1, DTP), jnp.float32).at[0, :DT].set(time_w)
    tb = jnp.zeros((1, DTP), jnp.float32).at[0, :DT].set(time_b)
    nmts_pad = jnp.pad(node_memory_ts, (0, NPAD - N))
    mbts_pad = jnp.pad(mailbox_ts, (0, NPAD - N))
    tsa_r = nmts_pad.reshape(TSR, 128)
    tsb_r = mbts_pad.reshape(TSR, 128)

    mem_g, mail_g, ts2 = _sc_gather(node_memory, mailbox, tsa_r, tsb_r,
                                    all_nodes)
    maxpos = _sc_winner(all_nodes)
    tsm = ts2[0].reshape(L, 1)
    tsb = ts2[1].reshape(L, 1)
    updated = _tc_gru(mem_g, mail_g, tsm, tsb, wm, wt, wh, bi, bh, tw, tb)
    mail_src, mail_dst = _tc_mail(updated, edge_feats)
    mts_g = ts2[1]

    nm_ref = jax.new_ref(node_memory)
    mb_ref = jax.new_ref(mailbox)
    nmts_ref = jax.new_ref(nmts_pad)
    mbts_ref = jax.new_ref(mbts_pad)
    _sc_cmem(nm_ref, nmts_ref, updated, mts_g, all_nodes)
    _sc_cmail(mb_ref, mbts_ref, mail_src, mail_dst, mts_g, all_nodes, maxpos)

    return (updated, nm_ref[...], nmts_ref[...][:N],
            mb_ref[...], mbts_ref[...][:N])


# speedup vs baseline: 4.3843x; 1.0116x over previous
import functools

import jax
import jax.numpy as jnp
from jax import lax
from jax.experimental import pallas as pl
from jax.experimental.pallas import tpu as pltpu
from jax.experimental.pallas import tpu_sc as plsc

N = 100000
L = 16384
E = L // 2
DM = 256
DE = 128
DMAIL = 2 * DM + DE
DT = 100
DTP = 128

NC = 2
NS = 16
NW = NC * NS
RPW = L // NW
GC = 32
SLICE = 3128
NPAD = NW * SLICE
TSR = NPAD // 128
LANES = 16
BMEM = 128
BMB = 48

_mesh = plsc.VectorSubcoreMesh(core_axis_name="c", subcore_axis_name="s")
_sc_params = pltpu.CompilerParams(needs_layout_passes=False)


def _worker_id():
    return lax.axis_index("s") * NC + lax.axis_index("c")


def _interleaved(an_v, j, h, lane):
    posv = 32 * j + 16 * h + lane
    half = lax.shift_right_logical(posv, 1)
    even = (posv & 1) == 0
    kidx = half + jnp.where(even, 0, E)
    nidv = plsc.load_gather(an_v, [kidx])
    return nidv, posv


def _sc_gather_body(nm_hbm, mb_hbm, tsa_hbm, tsb_hbm, an_hbm,
                    mem_g, mail_g, ts2,
                    an_v, rowidx_v, mem_b0, mem_b1, mail_b0, mail_b1,
                    ts_b, ts_stage,
                    sem_g0, sem_g1, sem_o0, sem_o1, sem_ts):
    wid = _worker_id()
    base = wid * RPW
    lane = lax.iota(jnp.int32, LANES)

    pltpu.sync_copy(an_hbm, an_v)

    def rowbuild(g, _):
        ids = an_v[pl.ds(base + g * LANES, LANES)]
        rowidx_v[pl.ds(g * LANES, LANES)] = lax.shift_right_logical(ids, 7)
        return 0

    lax.fori_loop(0, RPW // LANES, rowbuild, 0)

    HALF = 256
    for ti, tref in ((0, tsa_hbm), (1, tsb_hbm)):
        for hf in (0, 1):
            pltpu.async_copy(tref.at[rowidx_v.at[pl.ds(hf * HALF, HALF)]],
                             ts_b, sem_ts).wait()

            def extract(g, _):
                ids = an_v[pl.ds(base + hf * HALF + g * LANES, LANES)]
                colv = ids & 127
                v = plsc.load_gather(ts_b, [g * LANES + lane, colv])
                ts_stage[ti, pl.ds(hf * HALF + g * LANES, LANES)] = v
                return 0

            lax.fori_loop(0, HALF // LANES, extract, 0)

    pltpu.sync_copy(ts_stage, ts2.at[:, pl.ds(base, RPW)])

    mem_bufs = (mem_b0, mem_b1)
    mail_bufs = (mail_b0, mail_b1)
    sems_g = (sem_g0, sem_g1)
    sems_o = (sem_o0, sem_o1)
    nchunks = RPW // GC

    def issue(t, par):
        off = base + t * GC
        idx = an_v.at[pl.ds(off, GC)]
        pltpu.async_copy(nm_hbm.at[idx], mem_bufs[par], sems_g[par])
        pltpu.async_copy(mb_hbm.at[idx], mail_bufs[par], sems_g[par])

    for par in (0, 1):
        issue(par, par)

    def chunk(tp, _):
        for par in (0, 1):
            t = 2 * tp + par
            off = base + t * GC
            idx = an_v.at[pl.ds(off, GC)]
            pltpu.make_async_copy(nm_hbm.at[idx], mem_bufs[par],
                                  sems_g[par]).wait()
            pltpu.make_async_copy(mb_hbm.at[idx], mail_bufs[par],
                                  sems_g[par]).wait()
            cm = pltpu.async_copy(mem_bufs[par], mem_g.at[pl.ds(off, GC)],
                                  sems_o[par])
            cb = pltpu.async_copy(mail_bufs[par], mail_g.at[pl.ds(off, GC)],
                                  sems_o[par])
            cm.wait()
            cb.wait()

            @pl.when(t + 2 < nchunks)
            def _():
                issue(t + 2, par)

        return 0

    lax.fori_loop(0, nchunks // 2, chunk, 0)


@functools.partial(
    pl.kernel,
    out_type=(
        jax.ShapeDtypeStruct((L, DM), jnp.float32),
        jax.ShapeDtypeStruct((L, DMAIL), jnp.float32),
        jax.ShapeDtypeStruct((2, L), jnp.float32),
    ),
    mesh=_mesh,
    compiler_params=_sc_params,
    scratch_types=[
        pltpu.VMEM((L,), jnp.int32),
        pltpu.VMEM((RPW,), jnp.int32),
        pltpu.VMEM((GC, DM), jnp.float32),
        pltpu.VMEM((GC, DM), jnp.float32),
        pltpu.VMEM((GC, DMAIL), jnp.float32),
        pltpu.VMEM((GC, DMAIL), jnp.float32),
        pltpu.VMEM((256, 128), jnp.float32),
        pltpu.VMEM((2, RPW), jnp.float32),
        pltpu.SemaphoreType.DMA,
        pltpu.SemaphoreType.DMA,
        pltpu.SemaphoreType.DMA,
        pltpu.SemaphoreType.DMA,
        pltpu.SemaphoreType.DMA,
    ],
)
def _sc_gather(*refs):
    _sc_gather_body(*refs)


def _sc_winner_body(an_hbm, maxpos, an_v, tab_v, sh_v):
    wid = _worker_id()
    wlo = wid * SLICE
    lane = lax.iota(jnp.int32, LANES)

    pltpu.sync_copy(an_hbm, an_v)

    def clear(i, _):
        tab_v[pl.ds(i * LANES, LANES)] = jnp.full((LANES,), -1, jnp.int32)
        return 0

    lax.fori_loop(0, SLICE // LANES + 1, clear, 0)

    def scan(j, _):
        for h in (0, 1):
            nidv, posv = _interleaved(an_v, j, h, lane)
            key = (nidv << 14) | posv
            sk, sv = plsc.sort_key_val(key, posv)
            nid_s = lax.shift_right_logical(sk, 14)
            sh_v[...] = nid_s
            nxt = plsc.load_gather(sh_v, [jnp.minimum(lane + 1, LANES - 1)])
            is_end = (lane == LANES - 1) | (nid_s != nxt)
            m = is_end & (nid_s >= wlo) & (nid_s < wlo + SLICE)
            plsc.store_scatter(tab_v, [nid_s - wlo], sv, mask=m)
        return 0

    lax.fori_loop(0, L // 32, scan, 0)
    pltpu.sync_copy(tab_v.at[pl.ds(0, SLICE)], maxpos.at[pl.ds(wlo, SLICE)])


@functools.partial(
    pl.kernel,
    out_type=jax.ShapeDtypeStruct((NPAD,), jnp.int32),
    mesh=_mesh,
    compiler_params=_sc_params,
    scratch_types=[
        pltpu.VMEM((L,), jnp.int32),
        pltpu.VMEM((SLICE + LANES,), jnp.int32),
        pltpu.VMEM((LANES,), jnp.int32),
    ],
)
def _sc_winner(*refs):
    _sc_winner_body(*refs)


def _gru_body(mem_ref, mail_ref, tsm_ref, tsb_ref, wm_ref, wt_ref, wh_ref,
              bi_ref, bh_ref, tw_ref, tb_ref, upd_ref):
    mem = mem_ref[...]
    dt = tsb_ref[...] - tsm_ref[...]
    tf = jnp.cos(dt * tw_ref[...] + tb_ref[...])
    gi = (jnp.dot(mail_ref[...].astype(jnp.bfloat16), wm_ref[...],
                  preferred_element_type=jnp.float32)
          + jnp.dot(tf.astype(jnp.bfloat16), wt_ref[...],
                    preferred_element_type=jnp.float32)
          + bi_ref[...])
    gh = (jnp.dot(mem.astype(jnp.bfloat16), wh_ref[...],
                  preferred_element_type=jnp.float32) + bh_ref[...])
    r = jax.nn.sigmoid(gi[:, 0:DM] + gh[:, 0:DM])
    z = jax.nn.sigmoid(gi[:, DM:2 * DM] + gh[:, DM:2 * DM])
    n = jnp.tanh(gi[:, 2 * DM:] + r * gh[:, 2 * DM:])
    upd_ref[...] = (1.0 - z) * n + z * mem


def _tc_gru(mem_g, mail_g, tsm, tsb, wm, wt, wh, bi, bh, tw, tb):
    BM = 1024
    return pl.pallas_call(
        _gru_body,
        grid=(L // BM,),
        in_specs=[
            pl.BlockSpec((BM, DM), lambda i: (i, 0)),
            pl.BlockSpec((BM, DMAIL), lambda i: (i, 0)),
            pl.BlockSpec((BM, 1), lambda i: (i, 0)),
            pl.BlockSpec((BM, 1), lambda i: (i, 0)),
            pl.BlockSpec((DMAIL, 3 * DM), lambda i: (0, 0)),
            pl.BlockSpec((DTP, 3 * DM), lambda i: (0, 0)),
            pl.BlockSpec((DM, 3 * DM), lambda i: (0, 0)),
            pl.BlockSpec((1, 3 * DM), lambda i: (0, 0)),
            pl.BlockSpec((1, 3 * DM), lambda i: (0, 0)),
            pl.BlockSpec((1, DTP), lambda i: (0, 0)),
            pl.BlockSpec((1, DTP), lambda i: (0, 0)),
        ],
        out_specs=pl.BlockSpec((BM, DM), lambda i: (i, 0)),
        out_shape=jax.ShapeDtypeStruct((L, DM), jnp.float32),
    )(mem_g, mail_g, tsm, tsb, wm, wt, wh, bi, bh, tw, tb)


def _mail_body(us_ref, ud_ref, ef_ref, ms_ref, md_ref):
    s = us_ref[...]
    d = ud_ref[...]
    e = ef_ref[...]
    ms_ref[:, 0:DM] = s
    ms_ref[:, DM:2 * DM] = d
    ms_ref[:, 2 * DM:] = e
    md_ref[:, 0:DM] = d
    md_ref[:, DM:2 * DM] = s
    md_ref[:, 2 * DM:] = e


def _tc_mail(updated, edge_feats):
    BE = 1024
    nb = E // BE
    return pl.pallas_call(
        _mail_body,
        grid=(nb,),
        in_specs=[
            pl.BlockSpec((BE, DM), lambda i: (i, 0)),
            pl.BlockSpec((BE, DM), lambda i: (i + nb, 0)),
            pl.BlockSpec((BE, DE), lambda i: (i, 0)),
        ],
        out_specs=[
            pl.BlockSpec((BE, DMAIL), lambda i: (i, 0)),
            pl.BlockSpec((BE, DMAIL), lambda i: (i, 0)),
        ],
        out_shape=[
            jax.ShapeDtypeStruct((E, DMAIL), jnp.float32),
            jax.ShapeDtypeStruct((E, DMAIL), jnp.float32),
        ],
    )(updated, updated, edge_feats)


def _pad_list(lst, cnt, batch):
    @pl.when(cnt > 0)
    def _():
        rep = plsc.load_gather(lst, [jnp.zeros((LANES,), jnp.int32)])
        for i in range(batch // LANES):
            lst[pl.ds(cnt + i * LANES, LANES)] = rep


def _scatter_rows(lst, cnt, batch, src_hbm, dst_ref, an_v, off,
                  rbufs, ubufs, ebufs, sems_g, sems_s, via_edge):
    trips = (cnt + batch - 1) // batch

    def src_idx(t, par):
        if via_edge:
            return ebufs[par]
        return lst.at[pl.ds(t * batch, batch)]

    def prep_and_issue(t, par):
        for i in range(batch // LANES):
            v = lst[pl.ds(t * batch + i * LANES, LANES)]
            if via_edge:
                ev = lax.shift_right_logical(v, 1)
                ebufs[par][pl.ds(i * LANES, LANES)] = ev
            else:
                ev = v
            uv = plsc.load_gather(an_v, [ev + off])
            ubufs[par][pl.ds(i * LANES, LANES)] = uv
        pltpu.async_copy(src_hbm.at[src_idx(t, par)], rbufs[par], sems_g[par])

    @pl.when(trips > 0)
    def _():
        prep_and_issue(0, 0)

    def body(tp, _):
        for par in (0, 1):
            t = 2 * tp + par

            @pl.when(t + 1 < trips)
            def _():
                prep_and_issue(t + 1, 1 - par)

            @pl.when(t < trips)
            def _():
                pltpu.make_async_copy(src_hbm.at[src_idx(t, par)],
                                      rbufs[par], sems_g[par]).wait()
                pltpu.async_copy(rbufs[par], dst_ref.at[ubufs[par]],
                                 sems_s[par]).wait()

        return 0

    lax.fori_loop(0, (trips + 1) // 2, body, 0)


def _scatter_mail(lst, cnt, upd_hbm, ef_hbm, mb_ref, an_v, off,
                  rbas, rbbs, rbes, ubufs, iabufs, ibbufs, iebufs,
                  sems_g, sems_s):
    trips = (cnt + BMB - 1) // BMB

    def prep_and_issue(t, par):
        for i in range(BMB // LANES):
            v = lst[pl.ds(t * BMB + i * LANES, LANES)]
            ev = lax.shift_right_logical(v, 1)
            iebufs[par][pl.ds(i * LANES, LANES)] = ev
            iabufs[par][pl.ds(i * LANES, LANES)] = ev + off
            ibbufs[par][pl.ds(i * LANES, LANES)] = ev + (E - off)
            uv = plsc.load_gather(an_v, [ev + off])
            ubufs[par][pl.ds(i * LANES, LANES)] = uv
        pltpu.async_copy(upd_hbm.at[iabufs[par]], rbas[par], sems_g[par])
        pltpu.async_copy(upd_hbm.at[ibbufs[par]], rbbs[par], sems_g[par])
        pltpu.async_copy(ef_hbm.at[iebufs[par]], rbes[par], sems_g[par])

    @pl.when(trips > 0)
    def _():
        prep_and_issue(0, 0)

    def body(tp, _):
        for par in (0, 1):
            t = 2 * tp + par

            @pl.when(t + 1 < trips)
            def _():
                prep_and_issue(t + 1, 1 - par)

            @pl.when(t < trips)
            def _():
                pltpu.make_async_copy(upd_hbm.at[iabufs[par]], rbas[par],
                                      sems_g[par]).wait()
                pltpu.make_async_copy(upd_hbm.at[ibbufs[par]], rbbs[par],
                                      sems_g[par]).wait()
                pltpu.make_async_copy(ef_hbm.at[iebufs[par]], rbes[par],
                                      sems_g[par]).wait()
                ca = pltpu.async_copy(
                    rbas[par], mb_ref.at[ubufs[par], pl.ds(0, DM)],
                    sems_s[par])
                cb = pltpu.async_copy(
                    rbbs[par], mb_ref.at[ubufs[par], pl.ds(DM, DM)],
                    sems_s[par])
                ce = pltpu.async_copy(
                    rbes[par], mb_ref.at[ubufs[par], pl.ds(2 * DM, DE)],
                    sems_s[par])
                ca.wait()
                cb.wait()
                ce.wait()

        return 0

    lax.fori_loop(0, (trips + 1) // 2, body, 0)


def _sc_cmem_body(nm_ref, nmts_ref, upd_hbm, mts_hbm, an_hbm,
                  an_v, mts_v, nmts_v, memk,
                  rb0, rb1, iu0, iu1,
                  sem_g0, sem_g1, sem_s0, sem_s1):
    wid = _worker_id()
    wlo = wid * SLICE
    whi = wlo + SLICE
    lane = lax.iota(jnp.int32, LANES)

    pltpu.sync_copy(an_hbm, an_v)
    pltpu.sync_copy(mts_hbm, mts_v)
    pltpu.sync_copy(nmts_ref.at[pl.ds(wlo, SLICE)], nmts_v)

    def scan_mem(j, cnt):
        kv = j * LANES + lane
        u = an_v[pl.ds(j * LANES, LANES)]
        m = (u >= wlo) & (u < whi)
        plsc.store_compressed(memk.at[pl.ds(cnt, LANES)], kv, mask=m)
        return cnt + plsc.all_reduce_population_count(m)[0]

    cnt_k = lax.fori_loop(0, L // LANES, scan_mem, jnp.int32(0))
    _pad_list(memk, cnt_k, BMEM)

    def ts_mem(t, _):
        kv = memk[pl.ds(t * LANES, LANES)]
        uv = plsc.load_gather(an_v, [kv])
        tv = plsc.load_gather(mts_v, [kv])
        plsc.store_scatter(nmts_v, [uv - wlo], tv)
        return 0

    lax.fori_loop(0, (cnt_k + LANES - 1) // LANES, ts_mem, 0)
    pltpu.sync_copy(nmts_v, nmts_ref.at[pl.ds(wlo, SLICE)])

    _scatter_rows(memk, cnt_k, BMEM, upd_hbm, nm_ref, an_v, 0,
                  (rb0, rb1), (iu0, iu1), None,
                  (sem_g0, sem_g1), (sem_s0, sem_s1), via_edge=False)


@functools.partial(
    pl.kernel,
    out_type=(),
    mesh=_mesh,
    compiler_params=_sc_params,
    scratch_types=[
        pltpu.VMEM((L,), jnp.int32),
        pltpu.VMEM((L,), jnp.float32),
        pltpu.VMEM((SLICE,), jnp.float32),
        pltpu.VMEM((L + BMEM,), jnp.int32),
        pltpu.VMEM((BMEM, DM), jnp.float32),
        pltpu.VMEM((BMEM, DM), jnp.float32),
        pltpu.VMEM((BMEM,), jnp.int32),
        pltpu.VMEM((BMEM,), jnp.int32),
        pltpu.SemaphoreType.DMA,
        pltpu.SemaphoreType.DMA,
        pltpu.SemaphoreType.DMA,
        pltpu.SemaphoreType.DMA,
    ],
)
def _sc_cmem(*refs):
    _sc_cmem_body(*refs)


def _sc_cmail_body(mb_ref, mbts_ref, upd_hbm, ef_hbm, mts_hbm, an_hbm,
                   maxpos_hbm,
                   an_v, mts_v, tab_v, mbts_v, pe, po,
                   rba0, rba1, rbb0, rbb1, rbe0, rbe1,
                   iu0, iu1, ia0, ia1, ib0, ib1, ie0, ie1,
                   sem_g0, sem_g1, sem_s0, sem_s1):
    wid = _worker_id()
    wlo = wid * SLICE
    whi = wlo + SLICE
    lane = lax.iota(jnp.int32, LANES)

    pltpu.sync_copy(an_hbm, an_v)
    pltpu.sync_copy(mts_hbm, mts_v)
    pltpu.sync_copy(maxpos_hbm.at[pl.ds(wlo, SLICE)], tab_v)
    pltpu.sync_copy(mbts_ref.at[pl.ds(wlo, SLICE)], mbts_v)

    def scan_win(j, cnts):
        ce, co = cnts
        for h in (0, 1):
            nidv, posv = _interleaved(an_v, j, h, lane)
            inr = (nidv >= wlo) & (nidv < whi)
            t = plsc.load_gather(tab_v, [jnp.where(inr, nidv - wlo, 0)])
            win = inr & (t == posv)
            even = (posv & 1) == 0
            me = win & even
            mo = win & (~even)
            plsc.store_compressed(pe.at[pl.ds(ce, LANES)], posv, mask=me)
            ce = ce + plsc.all_reduce_population_count(me)[0]
            plsc.store_compressed(po.at[pl.ds(co, LANES)], posv, mask=mo)
            co = co + plsc.all_reduce_population_count(mo)[0]
        return (ce, co)

    cnt_e, cnt_o = lax.fori_loop(0, L // 32, scan_win,
                                 (jnp.int32(0), jnp.int32(0)))
    _pad_list(pe, cnt_e, BMB)
    _pad_list(po, cnt_o, BMB)

    def ts_mail(lst, off):
        def body(t, _):
            pv = lst[pl.ds(t * LANES, LANES)]
            ev = lax.shift_right_logical(pv, 1)
            uv = plsc.load_gather(an_v, [ev + off])
            tv = plsc.load_gather(mts_v, [pv])
            plsc.store_scatter(mbts_v, [uv - wlo], tv)
            return 0
        return body

    lax.fori_loop(0, (cnt_e + LANES - 1) // LANES, ts_mail(pe, 0), 0)
    lax.fori_loop(0, (cnt_o + LANES - 1) // LANES, ts_mail(po, E), 0)
    pltpu.sync_copy(mbts_v, mbts_ref.at[pl.ds(wlo, SLICE)])

    _scatter_mail(pe, cnt_e, upd_hbm, ef_hbm, mb_ref, an_v, 0,
                  (rba0, rba1), (rbb0, rbb1), (rbe0, rbe1),
                  (iu0, iu1), (ia0, ia1), (ib0, ib1), (ie0, ie1),
                  (sem_g0, sem_g1), (sem_s0, sem_s1))
    _scatter_mail(po, cnt_o, upd_hbm, ef_hbm, mb_ref, an_v, E,
                  (rba0, rba1), (rbb0, rbb1), (rbe0, rbe1),
                  (iu0, iu1), (ia0, ia1), (ib0, ib1), (ie0, ie1),
                  (sem_g0, sem_g1), (sem_s0, sem_s1))


@functools.partial(
    pl.kernel,
    out_type=(),
    mesh=_mesh,
    compiler_params=_sc_params,
    scratch_types=[
        pltpu.VMEM((L,), jnp.int32),
        pltpu.VMEM((L,), jnp.float32),
        pltpu.VMEM((SLICE,), jnp.int32),
        pltpu.VMEM((SLICE,), jnp.float32),
        pltpu.VMEM((E + BMB,), jnp.int32),
        pltpu.VMEM((E + BMB,), jnp.int32),
        pltpu.VMEM((BMB, DM), jnp.float32),
        pltpu.VMEM((BMB, DM), jnp.float32),
        pltpu.VMEM((BMB, DM), jnp.float32),
        pltpu.VMEM((BMB, DM), jnp.float32),
        pltpu.VMEM((BMB, DE), jnp.float32),
        pltpu.VMEM((BMB, DE), jnp.float32),
        pltpu.VMEM((BMB,), jnp.int32),
        pltpu.VMEM((BMB,), jnp.int32),
        pltpu.VMEM((BMB,), jnp.int32),
        pltpu.VMEM((BMB,), jnp.int32),
        pltpu.VMEM((BMB,), jnp.int32),
        pltpu.VMEM((BMB,), jnp.int32),
        pltpu.VMEM((BMB,), jnp.int32),
        pltpu.VMEM((BMB,), jnp.int32),
        pltpu.SemaphoreType.DMA,
        pltpu.SemaphoreType.DMA,
        pltpu.SemaphoreType.DMA,
        pltpu.SemaphoreType.DMA,
    ],
)
def _sc_cmail(*refs):
    _sc_cmail_body(*refs)


def kernel(node_memory, node_memory_ts, mailbox, mailbox_ts, edge_feats,
           w_ih, w_hh, b_ih, b_hh, time_w, time_b, all_nodes):
    w_ih_t = w_ih.T
    wm = w_ih_t[:DMAIL].astype(jnp.bfloat16)
    wt = jnp.zeros((DTP, 3 * DM), jnp.float32).at[:DT].set(
        w_ih_t[DMAIL:]).astype(jnp.bfloat16)
    wh = w_hh.T.astype(jnp.bfloat16)
    bi = b_ih.reshape(1, -1)
    bh = b_hh.reshape(1, -1)
    tw = jnp.zeros((1, DTP), jnp.float32).at[0, :DT].set(time_w)
    tb = jnp.zeros((1, DTP), jnp.float32).at[0, :DT].set(time_b)
    nmts_pad = jnp.pad(node_memory_ts, (0, NPAD - N))
    mbts_pad = jnp.pad(mailbox_ts, (0, NPAD - N))
    tsa_r = nmts_pad.reshape(TSR, 128)
    tsb_r = mbts_pad.reshape(TSR, 128)

    mem_g, mail_g, ts2 = _sc_gather(node_memory, mailbox, tsa_r, tsb_r,
                                    all_nodes)
    maxpos = _sc_winner(all_nodes)
    tsm = ts2[0].reshape(L, 1)
    tsb = ts2[1].reshape(L, 1)
    updated = _tc_gru(mem_g, mail_g, tsm, tsb, wm, wt, wh, bi, bh, tw, tb)
    mts_g = ts2[1]

    nm_ref = jax.new_ref(node_memory)
    mb_ref = jax.new_ref(mailbox)
    nmts_ref = jax.new_ref(nmts_pad)
    mbts_ref = jax.new_ref(mbts_pad)
    _sc_cmem(nm_ref, nmts_ref, updated, mts_g, all_nodes)
    _sc_cmail(mb_ref, mbts_ref, updated, edge_feats, mts_g, all_nodes, maxpos)

    return (updated, nm_ref[...], nmts_ref[...][:N],
            mb_ref[...], mbts_ref[...][:N])


# speedup vs baseline: 4.6311x; 1.0563x over previous
import functools

import jax
import jax.numpy as jnp
from jax import lax
from jax.experimental import pallas as pl
from jax.experimental.pallas import tpu as pltpu
from jax.experimental.pallas import tpu_sc as plsc

N = 100000
L = 16384
E = L // 2
DM = 256
DE = 128
DMAIL = 2 * DM + DE
DT = 100
DTP = 128

NC = 2
NS = 16
NW = NC * NS
RPW = L // NW
GC = 32
SLICE = 3128
NPAD = NW * SLICE
TSR = NPAD // 128
LANES = 16
BMEM = 128
BMB = 48

_mesh = plsc.VectorSubcoreMesh(core_axis_name="c", subcore_axis_name="s")
_sc_params = pltpu.CompilerParams(needs_layout_passes=False)


def _worker_id():
    return lax.axis_index("s") * NC + lax.axis_index("c")


def _interleaved(an_v, j, h, lane):
    posv = 32 * j + 16 * h + lane
    half = lax.shift_right_logical(posv, 1)
    even = (posv & 1) == 0
    kidx = half + jnp.where(even, 0, E)
    nidv = plsc.load_gather(an_v, [kidx])
    return nidv, posv


def _sc_gather_body(nm_hbm, mb_hbm, tsa_hbm, tsb_hbm, an_hbm,
                    mem_g, mail_g, ts2,
                    an_v, rowidx_v, mem_b0, mem_b1, mail_b0, mail_b1,
                    ts_b, ts_stage,
                    sem_g0, sem_g1, sem_o0, sem_o1, sem_ts):
    wid = _worker_id()
    base = wid * RPW
    lane = lax.iota(jnp.int32, LANES)

    pltpu.sync_copy(an_hbm, an_v)

    def rowbuild(g, _):
        ids = an_v[pl.ds(base + g * LANES, LANES)]
        rowidx_v[pl.ds(g * LANES, LANES)] = lax.shift_right_logical(ids, 7)
        return 0

    lax.fori_loop(0, RPW // LANES, rowbuild, 0)

    HALF = 256
    for ti, tref in ((0, tsa_hbm), (1, tsb_hbm)):
        for hf in (0, 1):
            pltpu.async_copy(tref.at[rowidx_v.at[pl.ds(hf * HALF, HALF)]],
                             ts_b, sem_ts).wait()

            def extract(g, _):
                ids = an_v[pl.ds(base + hf * HALF + g * LANES, LANES)]
                colv = ids & 127
                v = plsc.load_gather(ts_b, [g * LANES + lane, colv])
                ts_stage[ti, pl.ds(hf * HALF + g * LANES, LANES)] = v
                return 0

            lax.fori_loop(0, HALF // LANES, extract, 0)

    pltpu.sync_copy(ts_stage, ts2.at[:, pl.ds(base, RPW)])

    mem_bufs = (mem_b0, mem_b1)
    mail_bufs = (mail_b0, mail_b1)
    sems_g = (sem_g0, sem_g1)
    sems_o = (sem_o0, sem_o1)
    nchunks = RPW // GC

    def issue(t, par):
        off = base + t * GC
        idx = an_v.at[pl.ds(off, GC)]
        pltpu.async_copy(nm_hbm.at[idx], mem_bufs[par], sems_g[par])
        pltpu.async_copy(mb_hbm.at[idx], mail_bufs[par], sems_g[par])

    for par in (0, 1):
        issue(par, par)

    def chunk(tp, _):
        for par in (0, 1):
            t = 2 * tp + par
            off = base + t * GC
            idx = an_v.at[pl.ds(off, GC)]
            pltpu.make_async_copy(nm_hbm.at[idx], mem_bufs[par],
                                  sems_g[par]).wait()
            pltpu.make_async_copy(mb_hbm.at[idx], mail_bufs[par],
                                  sems_g[par]).wait()
            cm = pltpu.async_copy(mem_bufs[par], mem_g.at[pl.ds(off, GC)],
                                  sems_o[par])
            cb = pltpu.async_copy(mail_bufs[par], mail_g.at[pl.ds(off, GC)],
                                  sems_o[par])
            cm.wait()
            cb.wait()

            @pl.when(t + 2 < nchunks)
            def _():
                issue(t + 2, par)

        return 0

    lax.fori_loop(0, nchunks // 2, chunk, 0)


@functools.partial(
    pl.kernel,
    out_type=(
        jax.ShapeDtypeStruct((L, DM), jnp.float32),
        jax.ShapeDtypeStruct((L, DMAIL), jnp.float32),
        jax.ShapeDtypeStruct((2, L), jnp.float32),
    ),
    mesh=_mesh,
    compiler_params=_sc_params,
    scratch_types=[
        pltpu.VMEM((L,), jnp.int32),
        pltpu.VMEM((RPW,), jnp.int32),
        pltpu.VMEM((GC, DM), jnp.float32),
        pltpu.VMEM((GC, DM), jnp.float32),
        pltpu.VMEM((GC, DMAIL), jnp.float32),
        pltpu.VMEM((GC, DMAIL), jnp.float32),
        pltpu.VMEM((256, 128), jnp.float32),
        pltpu.VMEM((2, RPW), jnp.float32),
        pltpu.SemaphoreType.DMA,
        pltpu.SemaphoreType.DMA,
        pltpu.SemaphoreType.DMA,
        pltpu.SemaphoreType.DMA,
        pltpu.SemaphoreType.DMA,
    ],
)
def _sc_gather(*refs):
    _sc_gather_body(*refs)


def _sc_winner_body(an_hbm, maxpos, an_v, tab_v, sh_v):
    wid = _worker_id()
    wlo = wid * SLICE
    lane = lax.iota(jnp.int32, LANES)

    pltpu.sync_copy(an_hbm, an_v)

    def clear(i, _):
        tab_v[pl.ds(i * LANES, LANES)] = jnp.full((LANES,), -1, jnp.int32)
        return 0

    lax.fori_loop(0, SLICE // LANES + 1, clear, 0)

    def scan(j, _):
        for h in (0, 1):
            nidv, posv = _interleaved(an_v, j, h, lane)
            key = (nidv << 14) | posv
            sk, sv = plsc.sort_key_val(key, posv)
            nid_s = lax.shift_right_logical(sk, 14)
            sh_v[...] = nid_s
            nxt = plsc.load_gather(sh_v, [jnp.minimum(lane + 1, LANES - 1)])
            is_end = (lane == LANES - 1) | (nid_s != nxt)
            m = is_end & (nid_s >= wlo) & (nid_s < wlo + SLICE)
            plsc.store_scatter(tab_v, [nid_s - wlo], sv, mask=m)
        return 0

    lax.fori_loop(0, L // 32, scan, 0)
    pltpu.sync_copy(tab_v.at[pl.ds(0, SLICE)], maxpos.at[pl.ds(wlo, SLICE)])


@functools.partial(
    pl.kernel,
    out_type=jax.ShapeDtypeStruct((NPAD,), jnp.int32),
    mesh=_mesh,
    compiler_params=_sc_params,
    scratch_types=[
        pltpu.VMEM((L,), jnp.int32),
        pltpu.VMEM((SLICE + LANES,), jnp.int32),
        pltpu.VMEM((LANES,), jnp.int32),
    ],
)
def _sc_winner(*refs):
    _sc_winner_body(*refs)


def _gru_body(mem_ref, mail_ref, tsm_ref, tsb_ref, wm_ref, wt_ref, wh_ref,
              bi_ref, bh_ref, tw_ref, tb_ref, upd_ref):
    mem = mem_ref[...]
    dt = tsb_ref[...] - tsm_ref[...]
    tf = jnp.cos(dt * tw_ref[...] + tb_ref[...])
    gi = (jnp.dot(mail_ref[...].astype(jnp.bfloat16), wm_ref[...],
                  preferred_element_type=jnp.float32)
          + jnp.dot(tf.astype(jnp.bfloat16), wt_ref[...],
                    preferred_element_type=jnp.float32)
          + bi_ref[...])
    gh = (jnp.dot(mem.astype(jnp.bfloat16), wh_ref[...],
                  preferred_element_type=jnp.float32) + bh_ref[...])
    r = jax.nn.sigmoid(gi[:, 0:DM] + gh[:, 0:DM])
    z = jax.nn.sigmoid(gi[:, DM:2 * DM] + gh[:, DM:2 * DM])
    n = jnp.tanh(gi[:, 2 * DM:] + r * gh[:, 2 * DM:])
    upd_ref[...] = (1.0 - z) * n + z * mem


def _tc_gru(mem_g, mail_g, tsm, tsb, wm, wt, wh, bi, bh, tw, tb):
    BM = 1024
    return pl.pallas_call(
        _gru_body,
        grid=(L // BM,),
        in_specs=[
            pl.BlockSpec((BM, DM), lambda i: (i, 0)),
            pl.BlockSpec((BM, DMAIL), lambda i: (i, 0)),
            pl.BlockSpec((BM, 1), lambda i: (i, 0)),
            pl.BlockSpec((BM, 1), lambda i: (i, 0)),
            pl.BlockSpec((DMAIL, 3 * DM), lambda i: (0, 0)),
            pl.BlockSpec((DTP, 3 * DM), lambda i: (0, 0)),
            pl.BlockSpec((DM, 3 * DM), lambda i: (0, 0)),
            pl.BlockSpec((1, 3 * DM), lambda i: (0, 0)),
            pl.BlockSpec((1, 3 * DM), lambda i: (0, 0)),
            pl.BlockSpec((1, DTP), lambda i: (0, 0)),
            pl.BlockSpec((1, DTP), lambda i: (0, 0)),
        ],
        out_specs=pl.BlockSpec((BM, DM), lambda i: (i, 0)),
        out_shape=jax.ShapeDtypeStruct((L, DM), jnp.float32),
    )(mem_g, mail_g, tsm, tsb, wm, wt, wh, bi, bh, tw, tb)


def _mail_body(us_ref, ud_ref, ef_ref, ms_ref, md_ref):
    s = us_ref[...]
    d = ud_ref[...]
    e = ef_ref[...]
    ms_ref[:, 0:DM] = s
    ms_ref[:, DM:2 * DM] = d
    ms_ref[:, 2 * DM:] = e
    md_ref[:, 0:DM] = d
    md_ref[:, DM:2 * DM] = s
    md_ref[:, 2 * DM:] = e


def _tc_mail(updated, edge_feats):
    BE = 1024
    nb = E // BE
    return pl.pallas_call(
        _mail_body,
        grid=(nb,),
        in_specs=[
            pl.BlockSpec((BE, DM), lambda i: (i, 0)),
            pl.BlockSpec((BE, DM), lambda i: (i + nb, 0)),
            pl.BlockSpec((BE, DE), lambda i: (i, 0)),
        ],
        out_specs=[
            pl.BlockSpec((BE, DMAIL), lambda i: (i, 0)),
            pl.BlockSpec((BE, DMAIL), lambda i: (i, 0)),
        ],
        out_shape=[
            jax.ShapeDtypeStruct((E, DMAIL), jnp.float32),
            jax.ShapeDtypeStruct((E, DMAIL), jnp.float32),
        ],
    )(updated, updated, edge_feats)


def _pad_list(lst, cnt, batch):
    @pl.when(cnt > 0)
    def _():
        rep = plsc.load_gather(lst, [jnp.zeros((LANES,), jnp.int32)])
        for i in range(batch // LANES):
            lst[pl.ds(cnt + i * LANES, LANES)] = rep


def _scatter_rows(lst, cnt, batch, src_hbm, dst_ref, an_v, off,
                  rbufs, ubufs, ebufs, sems_g, sems_s, via_edge):
    trips = (cnt + batch - 1) // batch

    def src_idx(t, par):
        if via_edge:
            return ebufs[par]
        return lst.at[pl.ds(t * batch, batch)]

    def prep_and_issue(t, par):
        for i in range(batch // LANES):
            v = lst[pl.ds(t * batch + i * LANES, LANES)]
            if via_edge:
                ev = lax.shift_right_logical(v, 1)
                ebufs[par][pl.ds(i * LANES, LANES)] = ev
            else:
                ev = v
            uv = plsc.load_gather(an_v, [ev + off])
            ubufs[par][pl.ds(i * LANES, LANES)] = uv
        pltpu.async_copy(src_hbm.at[src_idx(t, par)], rbufs[par], sems_g[par])

    @pl.when(trips > 0)
    def _():
        prep_and_issue(0, 0)

    def body(tp, _):
        for par in (0, 1):
            t = 2 * tp + par

            @pl.when(t + 1 < trips)
            def _():
                prep_and_issue(t + 1, 1 - par)

            @pl.when(t < trips)
            def _():
                pltpu.make_async_copy(src_hbm.at[src_idx(t, par)],
                                      rbufs[par], sems_g[par]).wait()
                pltpu.async_copy(rbufs[par], dst_ref.at[ubufs[par]],
                                 sems_s[par]).wait()

        return 0

    lax.fori_loop(0, (trips + 1) // 2, body, 0)


def _scatter_mail(lst, cnt, upd_hbm, ef_hbm, mb_ref, an_v, off,
                  rbas, rbbs, rbes, ubufs, iabufs, ibbufs, iebufs,
                  sems_g, sems_s):
    trips = (cnt + BMB - 1) // BMB

    def prep_and_issue(t, par):
        for i in range(BMB // LANES):
            v = lst[pl.ds(t * BMB + i * LANES, LANES)]
            ev = lax.shift_right_logical(v, 1)
            iebufs[par][pl.ds(i * LANES, LANES)] = ev
            iabufs[par][pl.ds(i * LANES, LANES)] = ev + off
            ibbufs[par][pl.ds(i * LANES, LANES)] = ev + (E - off)
            uv = plsc.load_gather(an_v, [ev + off])
            ubufs[par][pl.ds(i * LANES, LANES)] = uv
        pltpu.async_copy(upd_hbm.at[iabufs[par]], rbas[par], sems_g[par])
        pltpu.async_copy(upd_hbm.at[ibbufs[par]], rbbs[par], sems_g[par])
        pltpu.async_copy(ef_hbm.at[iebufs[par]], rbes[par], sems_g[par])

    @pl.when(trips > 0)
    def _():
        prep_and_issue(0, 0)

    def body(tp, _):
        for par in (0, 1):
            t = 2 * tp + par

            @pl.when(t + 1 < trips)
            def _():
                prep_and_issue(t + 1, 1 - par)

            @pl.when(t < trips)
            def _():
                pltpu.make_async_copy(upd_hbm.at[iabufs[par]], rbas[par],
                                      sems_g[par]).wait()
                pltpu.make_async_copy(upd_hbm.at[ibbufs[par]], rbbs[par],
                                      sems_g[par]).wait()
                pltpu.make_async_copy(ef_hbm.at[iebufs[par]], rbes[par],
                                      sems_g[par]).wait()
                ca = pltpu.async_copy(
                    rbas[par], mb_ref.at[ubufs[par], pl.ds(0, DM)],
                    sems_s[par])
                cb = pltpu.async_copy(
                    rbbs[par], mb_ref.at[ubufs[par], pl.ds(DM, DM)],
                    sems_s[par])
                ce = pltpu.async_copy(
                    rbes[par], mb_ref.at[ubufs[par], pl.ds(2 * DM, DE)],
                    sems_s[par])
                ca.wait()
                cb.wait()
                ce.wait()

        return 0

    lax.fori_loop(0, (trips + 1) // 2, body, 0)


def _sc_cmem_body(nm_ref, nmts_ref, upd_hbm, mts_hbm, an_hbm,
                  an_v, mts_v, nmts_v, memk,
                  rb0, rb1, iu0, iu1,
                  sem_g0, sem_g1, sem_s0, sem_s1):
    wid = _worker_id()
    wlo = wid * SLICE
    whi = wlo + SLICE
    lane = lax.iota(jnp.int32, LANES)

    pltpu.sync_copy(an_hbm, an_v)
    pltpu.sync_copy(mts_hbm, mts_v)
    pltpu.sync_copy(nmts_ref.at[pl.ds(wlo, SLICE)], nmts_v)

    def scan_mem(j, cnt):
        kv = j * LANES + lane
        u = an_v[pl.ds(j * LANES, LANES)]
        m = (u >= wlo) & (u < whi)
        plsc.store_compressed(memk.at[pl.ds(cnt, LANES)], kv, mask=m)
        return cnt + plsc.all_reduce_population_count(m)[0]

    cnt_k = lax.fori_loop(0, L // LANES, scan_mem, jnp.int32(0))
    _pad_list(memk, cnt_k, BMEM)

    def ts_mem(t, _):
        kv = memk[pl.ds(t * LANES, LANES)]
        uv = plsc.load_gather(an_v, [kv])
        tv = plsc.load_gather(mts_v, [kv])
        plsc.store_scatter(nmts_v, [uv - wlo], tv)
        return 0

    lax.fori_loop(0, (cnt_k + LANES - 1) // LANES, ts_mem, 0)
    pltpu.sync_copy(nmts_v, nmts_ref.at[pl.ds(wlo, SLICE)])

    _scatter_rows(memk, cnt_k, BMEM, upd_hbm, nm_ref, an_v, 0,
                  (rb0, rb1), (iu0, iu1), None,
                  (sem_g0, sem_g1), (sem_s0, sem_s1), via_edge=False)


@functools.partial(
    pl.kernel,
    out_type=(),
    mesh=_mesh,
    compiler_params=_sc_params,
    scratch_types=[
        pltpu.VMEM((L,), jnp.int32),
        pltpu.VMEM((L,), jnp.float32),
        pltpu.VMEM((SLICE,), jnp.float32),
        pltpu.VMEM((L + BMEM,), jnp.int32),
        pltpu.VMEM((BMEM, DM), jnp.float32),
        pltpu.VMEM((BMEM, DM), jnp.float32),
        pltpu.VMEM((BMEM,), jnp.int32),
        pltpu.VMEM((BMEM,), jnp.int32),
        pltpu.SemaphoreType.DMA,
        pltpu.SemaphoreType.DMA,
        pltpu.SemaphoreType.DMA,
        pltpu.SemaphoreType.DMA,
    ],
)
def _sc_cmem(*refs):
    _sc_cmem_body(*refs)


def _sc_cmail_body(mb_ref, mbts_ref, upd_hbm, ef_hbm, mts_hbm, an_hbm,
                   maxpos_hbm,
                   an_v, mts_v, tab_v, mbts_v, pe, po,
                   rba0, rba1, rbb0, rbb1, rbe0, rbe1,
                   iu0, iu1, ia0, ia1, ib0, ib1, ie0, ie1,
                   sem_g0, sem_g1, sem_s0, sem_s1):
    wid = _worker_id()
    wlo = wid * SLICE
    whi = wlo + SLICE
    lane = lax.iota(jnp.int32, LANES)

    pltpu.sync_copy(an_hbm, an_v)
    pltpu.sync_copy(mts_hbm, mts_v)
    pltpu.sync_copy(maxpos_hbm.at[pl.ds(wlo, SLICE)], tab_v)
    pltpu.sync_copy(mbts_ref.at[pl.ds(wlo, SLICE)], mbts_v)

    def scan_win(j, cnts):
        ce, co = cnts
        for h in (0, 1):
            nidv, posv = _interleaved(an_v, j, h, lane)
            inr = (nidv >= wlo) & (nidv < whi)
            t = plsc.load_gather(tab_v, [jnp.where(inr, nidv - wlo, 0)])
            win = inr & (t == posv)
            even = (posv & 1) == 0
            me = win & even
            mo = win & (~even)
            plsc.store_compressed(pe.at[pl.ds(ce, LANES)], posv, mask=me)
            ce = ce + plsc.all_reduce_population_count(me)[0]
            plsc.store_compressed(po.at[pl.ds(co, LANES)], posv, mask=mo)
            co = co + plsc.all_reduce_population_count(mo)[0]
        return (ce, co)

    cnt_e, cnt_o = lax.fori_loop(0, L // 32, scan_win,
                                 (jnp.int32(0), jnp.int32(0)))
    _pad_list(pe, cnt_e, BMB)
    _pad_list(po, cnt_o, BMB)

    def ts_mail(lst, off):
        def body(t, _):
            pv = lst[pl.ds(t * LANES, LANES)]
            ev = lax.shift_right_logical(pv, 1)
            uv = plsc.load_gather(an_v, [ev + off])
            tv = plsc.load_gather(mts_v, [pv])
            plsc.store_scatter(mbts_v, [uv - wlo], tv)
            return 0
        return body

    lax.fori_loop(0, (cnt_e + LANES - 1) // LANES, ts_mail(pe, 0), 0)
    lax.fori_loop(0, (cnt_o + LANES - 1) // LANES, ts_mail(po, E), 0)
    pltpu.sync_copy(mbts_v, mbts_ref.at[pl.ds(wlo, SLICE)])

    _scatter_mail(pe, cnt_e, upd_hbm, ef_hbm, mb_ref, an_v, 0,
                  (rba0, rba1), (rbb0, rbb1), (rbe0, rbe1),
                  (iu0, iu1), (ia0, ia1), (ib0, ib1), (ie0, ie1),
                  (sem_g0, sem_g1), (sem_s0, sem_s1))
    _scatter_mail(po, cnt_o, upd_hbm, ef_hbm, mb_ref, an_v, E,
                  (rba0, rba1), (rbb0, rbb1), (rbe0, rbe1),
                  (iu0, iu1), (ia0, ia1), (ib0, ib1), (ie0, ie1),
                  (sem_g0, sem_g1), (sem_s0, sem_s1))


@functools.partial(
    pl.kernel,
    out_type=(),
    mesh=_mesh,
    compiler_params=_sc_params,
    scratch_types=[
        pltpu.VMEM((L,), jnp.int32),
        pltpu.VMEM((L,), jnp.float32),
        pltpu.VMEM((SLICE,), jnp.int32),
        pltpu.VMEM((SLICE,), jnp.float32),
        pltpu.VMEM((E + BMB,), jnp.int32),
        pltpu.VMEM((E + BMB,), jnp.int32),
        pltpu.VMEM((BMB, DM), jnp.float32),
        pltpu.VMEM((BMB, DM), jnp.float32),
        pltpu.VMEM((BMB, DM), jnp.float32),
        pltpu.VMEM((BMB, DM), jnp.float32),
        pltpu.VMEM((BMB, DE), jnp.float32),
        pltpu.VMEM((BMB, DE), jnp.float32),
        pltpu.VMEM((BMB,), jnp.int32),
        pltpu.VMEM((BMB,), jnp.int32),
        pltpu.VMEM((BMB,), jnp.int32),
        pltpu.VMEM((BMB,), jnp.int32),
        pltpu.VMEM((BMB,), jnp.int32),
        pltpu.VMEM((BMB,), jnp.int32),
        pltpu.VMEM((BMB,), jnp.int32),
        pltpu.VMEM((BMB,), jnp.int32),
        pltpu.SemaphoreType.DMA,
        pltpu.SemaphoreType.DMA,
        pltpu.SemaphoreType.DMA,
        pltpu.SemaphoreType.DMA,
    ],
)
def _sc_cmail(*refs):
    _sc_cmail_body(*refs)


def kernel(node_memory, node_memory_ts, mailbox, mailbox_ts, edge_feats,
           w_ih, w_hh, b_ih, b_hh, time_w, time_b, all_nodes):
    w_ih_t = w_ih.T
    wm = w_ih_t[:DMAIL].astype(jnp.bfloat16)
    wt = jnp.zeros((DTP, 3 * DM), jnp.float32).at[:DT].set(
        w_ih_t[DMAIL:]).astype(jnp.bfloat16)
    wh = w_hh.T.astype(jnp.bfloat16)
    bi = b_ih.reshape(1, -1)
    bh = b_hh.reshape(1, -1)
    tw = jnp.zeros((1, DTP), jnp.float32).at[0, :DT].set(time_w)
    tb = jnp.zeros((1, DTP), jnp.float32).at[0, :DT].set(time_b)
    nmts_pad = jnp.pad(node_memory_ts, (0, NPAD - N))
    mbts_pad = jnp.pad(mailbox_ts, (0, NPAD - N))
    tsa_r = nmts_pad.reshape(TSR, 128)
    tsb_r = mbts_pad.reshape(TSR, 128)

    mem_g, mail_g, ts2 = _sc_gather(node_memory, mailbox, tsa_r, tsb_r,
                                    all_nodes)
    maxpos = _sc_winner(all_nodes)
    tsm = ts2[0].reshape(L, 1)
    tsb = ts2[1].reshape(L, 1)
    updated = _tc_gru(mem_g, mail_g, tsm, tsb, wm, wt, wh, bi, bh, tw, tb)
    mts_g = ts2[1]

    dep = (tsa_r[0, 0] + tsb_r[0, 0]) * 0.0
    nm_ref = jax.new_ref(node_memory + dep)
    mb_ref = jax.new_ref(mailbox + dep)
    nmts_ref = jax.new_ref(nmts_pad)
    mbts_ref = jax.new_ref(mbts_pad)
    _sc_cmem(nm_ref, nmts_ref, updated, mts_g, all_nodes)
    _sc_cmail(mb_ref, mbts_ref, updated, edge_feats, mts_g, all_nodes, maxpos)

    return (updated, nm_ref[...], nmts_ref[...][:N],
            mb_ref[...], mbts_ref[...][:N])
